# Initial kernel scaffold; baseline (speedup 1.0000x reference)
import jax
import jax.numpy as jnp
from jax.experimental import pallas as pl


def kernel(x, edge_index, batch, set_batch, W1, b1, W2, b2, W3, b3, psi_W1, psi_b1, psi_W2, psi_b2, phi_W1, phi_b1, phi_W2, phi_b2):
    raise NotImplementedError("write your pallas kernel here")


# speedup vs baseline: 18.8240x; 18.8240x over previous
import functools

import jax
import jax.numpy as jnp
from jax import lax
from jax.experimental import pallas as pl
from jax.experimental.pallas import tpu as pltpu
from jax.experimental.pallas import tpu_sc as plsc

N = 10000
NP = 10240
E = 320000
D = 128
NG = 512
NS = 64
NTILES = 32
NCH = 79
CH = 128
RB = 1280

_mesh = plsc.VectorSubcoreMesh(core_axis_name="c", subcore_axis_name="s")


@functools.partial(
    pl.kernel,
    out_type=jax.ShapeDtypeStruct((2, NP), jnp.float32),
    mesh=_mesh,
    scratch_types=[
        pltpu.VMEM((NCH, CH), jnp.int32),
        pltpu.VMEM((CH,), jnp.float32),
        pltpu.VMEM_SHARED((NP,), jnp.float32),
    ],
)
def _deg_kernel(colc_hbm, ones_hbm, z1_hbm, out_hbm, colidx_v, ones_v, acc_sh):
    c = lax.axis_index("c")
    s = lax.axis_index("s")
    wid = s * 2 + c
    sl = 640
    pltpu.sync_copy(ones_hbm, ones_v)
    pltpu.sync_copy(z1_hbm.at[pl.ds(s * sl, sl)], acc_sh.at[pl.ds(s * sl, sl)])
    pltpu.sync_copy(colc_hbm.at[wid], colidx_v)
    plsc.subcore_barrier()

    def body(j, carry):
        pltpu.sync_copy(ones_v, acc_sh.at[colidx_v.at[j]], add=True)
        return carry

    lax.fori_loop(0, NCH, body, 0)
    plsc.subcore_barrier()
    pltpu.sync_copy(acc_sh.at[pl.ds(s * sl, sl)], out_hbm.at[c, pl.ds(s * sl, sl)])


@functools.partial(
    pl.kernel,
    out_type=jax.ShapeDtypeStruct((2, NP, D), jnp.float32),
    mesh=_mesh,
    scratch_types=[
        pltpu.VMEM((NCH, CH), jnp.int32),
        pltpu.VMEM((NCH, CH), jnp.int32),
        pltpu.VMEM((CH, D), jnp.float32),
        pltpu.SemaphoreType.DMA,
        pltpu.VMEM_SHARED((NP, D), jnp.float32),
    ],
)
def _edge_kernel(hhat_hbm, z2_hbm, rowc_hbm, colc_hbm, out_hbm,
                 rowidx_v, colidx_v, rows_v, sem, acc_sh):
    c = lax.axis_index("c")
    s = lax.axis_index("s")
    wid = s * 2 + c
    sl = 640

    @pl.when(c == 0)
    def _():
        pltpu.sync_copy(hhat_hbm.at[pl.ds(s * sl, sl)],
                        acc_sh.at[pl.ds(s * sl, sl)])

    @pl.when(c == 1)
    def _():
        pltpu.sync_copy(z2_hbm.at[pl.ds(s * sl, sl)],
                        acc_sh.at[pl.ds(s * sl, sl)])

    pltpu.sync_copy(rowc_hbm.at[wid], rowidx_v)
    pltpu.sync_copy(colc_hbm.at[wid], colidx_v)
    plsc.subcore_barrier()

    def body(j, carry):
        pltpu.async_copy(hhat_hbm.at[rowidx_v.at[j]], rows_v, sem).wait()
        pltpu.sync_copy(rows_v, acc_sh.at[colidx_v.at[j]], add=True)
        return carry

    lax.fori_loop(0, NCH, body, 0)
    plsc.subcore_barrier()
    pltpu.sync_copy(acc_sh.at[pl.ds(s * sl, sl)], out_hbm.at[c, pl.ds(s * sl, sl)])


def _pre_body(x_ref, da_ref, db_ref, w_ref, hh_ref, dinv_ref):
    dinv = lax.rsqrt(da_ref[...] + db_ref[...] + 1.0)
    hh_ref[...] = jnp.dot(x_ref[...], w_ref[...],
                          preferred_element_type=jnp.float32) * dinv
    dinv_ref[...] = dinv


_pre = pl.pallas_call(
    _pre_body,
    grid=(NP // RB,),
    in_specs=[
        pl.BlockSpec((RB, D), lambda i: (i, 0)),
        pl.BlockSpec((RB, 1), lambda i: (i, 0)),
        pl.BlockSpec((RB, 1), lambda i: (i, 0)),
        pl.BlockSpec((D, D), lambda i: (0, 0)),
    ],
    out_specs=[
        pl.BlockSpec((RB, D), lambda i: (i, 0)),
        pl.BlockSpec((RB, 1), lambda i: (i, 0)),
    ],
    out_shape=[
        jax.ShapeDtypeStruct((NP, D), jnp.float32),
        jax.ShapeDtypeStruct((NP, 1), jnp.float32),
    ],
)


def _mid_body(a0_ref, a1_ref, dinv_ref, b_ref, w_ref, out_ref):
    dinv = dinv_ref[...]
    h = jnp.maximum(dinv * (a0_ref[...] + a1_ref[...]) + b_ref[...], 0.0)
    out_ref[...] = jnp.dot(h, w_ref[...],
                           preferred_element_type=jnp.float32) * dinv


_mid = pl.pallas_call(
    _mid_body,
    grid=(NP // RB,),
    in_specs=[
        pl.BlockSpec((RB, D), lambda i: (i, 0)),
        pl.BlockSpec((RB, D), lambda i: (i, 0)),
        pl.BlockSpec((RB, 1), lambda i: (i, 0)),
        pl.BlockSpec((1, D), lambda i: (0, 0)),
        pl.BlockSpec((D, D), lambda i: (0, 0)),
    ],
    out_specs=pl.BlockSpec((RB, D), lambda i: (i, 0)),
    out_shape=jax.ShapeDtypeStruct((NP, D), jnp.float32),
)


def _fin_body(a0_ref, a1_ref, dinv_ref, b3_ref, batch_ref, sb_ref,
              psw1_ref, psb1_ref, psw2_ref, psb2_ref,
              phw1_ref, phb1_ref, phw2_ref, phb2_ref,
              y_ref, pooled, cnt):
    j = pl.program_id(0)

    @pl.when(j == 0)
    def _():
        pooled[...] = jnp.zeros_like(pooled)
        cnt[...] = jnp.zeros_like(cnt)

    h = jnp.maximum(dinv_ref[...] * (a0_ref[...] + a1_ref[...]) + b3_ref[...],
                    0.0)
    gids = lax.broadcasted_iota(jnp.int32, (NG, RB), 0)
    mf = (batch_ref[...] == gids).astype(jnp.float32)
    pooled[...] += jnp.dot(mf, h, preferred_element_type=jnp.float32)
    cnt[...] += jnp.sum(mf, axis=1, keepdims=True)

    @pl.when(j == NP // RB - 1)
    def _():
        emb = pooled[...] / jnp.maximum(cnt[...], 1.0)
        t = jnp.maximum(
            jnp.dot(emb, psw1_ref[...], preferred_element_type=jnp.float32)
            + psb1_ref[...], 0.0)
        p = jnp.tanh(
            jnp.dot(t, psw2_ref[...], preferred_element_type=jnp.float32)
            + psb2_ref[...])
        p0 = jnp.tanh(
            jnp.dot(jnp.maximum(psb1_ref[...], 0.0), psw2_ref[...],
                    preferred_element_type=jnp.float32) + psb2_ref[...])
        sids = lax.broadcasted_iota(jnp.int32, (NS, NG), 0)
        sm = (sb_ref[...] == sids).astype(jnp.float32)
        ssize = jnp.sum(sm, axis=1, keepdims=True)
        mx = jnp.max(ssize)
        agg = jnp.dot(sm, p, preferred_element_type=jnp.float32) \
            + (mx - ssize) * p0
        t2 = jnp.maximum(
            jnp.dot(agg, phw1_ref[...], preferred_element_type=jnp.float32)
            + phb1_ref[...], 0.0)
        y_ref[...] = jnp.dot(t2, phw2_ref[...],
                             preferred_element_type=jnp.float32) + phb2_ref[...]


_fin = pl.pallas_call(
    _fin_body,
    grid=(NP // RB,),
    in_specs=[
        pl.BlockSpec((RB, D), lambda i: (i, 0)),
        pl.BlockSpec((RB, D), lambda i: (i, 0)),
        pl.BlockSpec((RB, 1), lambda i: (i, 0)),
        pl.BlockSpec((1, D), lambda i: (0, 0)),
        pl.BlockSpec((1, RB), lambda i: (0, i)),
        pl.BlockSpec((1, NG), lambda i: (0, 0)),
        pl.BlockSpec((D, D), lambda i: (0, 0)),
        pl.BlockSpec((1, D), lambda i: (0, 0)),
        pl.BlockSpec((D, D), lambda i: (0, 0)),
        pl.BlockSpec((1, D), lambda i: (0, 0)),
        pl.BlockSpec((D, D), lambda i: (0, 0)),
        pl.BlockSpec((1, D), lambda i: (0, 0)),
        pl.BlockSpec((D, 16), lambda i: (0, 0)),
        pl.BlockSpec((1, 16), lambda i: (0, 0)),
    ],
    out_specs=pl.BlockSpec((NS, 16), lambda i: (0, 0)),
    out_shape=jax.ShapeDtypeStruct((NS, 16), jnp.float32),
    scratch_shapes=[
        pltpu.VMEM((NG, D), jnp.float32),
        pltpu.VMEM((NG, 1), jnp.float32),
    ],
)


def kernel(x, edge_index, batch, set_batch, W1, b1, W2, b2, W3, b3,
           psi_W1, psi_b1, psi_W2, psi_b2, phi_W1, phi_b1, phi_W2, phi_b2):
    f32 = jnp.float32
    ep = NTILES * NCH * CH - E
    pad_row = (jnp.arange(ep, dtype=jnp.int32) % N)
    pad_col = N + (jnp.arange(ep, dtype=jnp.int32) % (NP - N))
    rowc = jnp.concatenate([edge_index[0], pad_row]).reshape(NTILES, NCH, CH)
    colc = jnp.concatenate([edge_index[1], pad_col]).reshape(NTILES, NCH, CH)
    ones = jnp.ones((CH,), f32)
    z1 = jnp.zeros((NP,), f32)
    z2 = jnp.zeros((NP, D), f32)
    xpad = jnp.concatenate([x, jnp.zeros((NP - N, D), f32)], axis=0)
    bpad = jnp.concatenate(
        [batch.astype(jnp.int32),
         jnp.full((NP - N,), NG, jnp.int32)]).reshape(1, NP)
    sb = set_batch.astype(jnp.int32).reshape(1, NG)

    degp = _deg_kernel(colc, ones, z1)
    da = degp[0].reshape(NP, 1)
    db = degp[1].reshape(NP, 1)
    hh, dinv = _pre(xpad, da, db, W1)

    part = _edge_kernel(hh, z2, rowc, colc)
    hh = _mid(part[0], part[1], dinv, b1.reshape(1, D), W2)
    part = _edge_kernel(hh, z2, rowc, colc)
    hh = _mid(part[0], part[1], dinv, b2.reshape(1, D), W3)
    part = _edge_kernel(hh, z2, rowc, colc)

    y = _fin(part[0], part[1], dinv, b3.reshape(1, D), bpad, sb,
             psi_W1, psi_b1.reshape(1, D), psi_W2, psi_b2.reshape(1, D),
             phi_W1, phi_b1.reshape(1, D), phi_W2, phi_b2.reshape(1, 16))
    return y


# speedup vs baseline: 28.7466x; 1.5271x over previous
import functools

import jax
import jax.numpy as jnp
from jax import lax
from jax.experimental import pallas as pl
from jax.experimental.pallas import tpu as pltpu
from jax.experimental.pallas import tpu_sc as plsc

N = 10000
NP = 10240
E = 320000
D = 128
NG = 512
NS = 64
NTILES = 32
NCH = 90
CH = 112
NBUF = 3
NGRP = 45
NRING = 4
RB = 1280

_mesh = plsc.VectorSubcoreMesh(core_axis_name="c", subcore_axis_name="s")


@functools.partial(
    pl.kernel,
    out_type=jax.ShapeDtypeStruct((2, NP), jnp.float32),
    mesh=_mesh,
    scratch_types=[
        pltpu.VMEM((NCH, CH), jnp.int32),
        pltpu.VMEM((CH,), jnp.float32),
        pltpu.VMEM_SHARED((NP,), jnp.float32),
    ],
)
def _deg_kernel(colc_hbm, ones_hbm, z1_hbm, out_hbm, colidx_v, ones_v, acc_sh):
    c = lax.axis_index("c")
    s = lax.axis_index("s")
    wid = s * 2 + c
    sl = 640
    pltpu.sync_copy(ones_hbm, ones_v)
    pltpu.sync_copy(z1_hbm.at[pl.ds(s * sl, sl)], acc_sh.at[pl.ds(s * sl, sl)])
    pltpu.sync_copy(colc_hbm.at[wid], colidx_v)
    plsc.subcore_barrier()

    def body(j, carry):
        pltpu.sync_copy(ones_v, acc_sh.at[colidx_v.at[j]], add=True)
        return carry

    lax.fori_loop(0, NCH, body, 0)
    plsc.subcore_barrier()
    pltpu.sync_copy(acc_sh.at[pl.ds(s * sl, sl)], out_hbm.at[c, pl.ds(s * sl, sl)])


@functools.partial(
    pl.kernel,
    out_type=jax.ShapeDtypeStruct((2, NP, D), jnp.float32),
    mesh=_mesh,
    scratch_types=[
        pltpu.VMEM((NRING, 2, CH), jnp.int32),
        pltpu.VMEM((NRING, 2, CH), jnp.int32),
        pltpu.VMEM((NBUF, CH, D), jnp.float32),
        pltpu.SemaphoreType.DMA((NRING,)),
        pltpu.SemaphoreType.DMA((NBUF,)),
        pltpu.SemaphoreType.DMA((NBUF,)),
        pltpu.VMEM_SHARED((NP, D), jnp.float32),
    ],
)
def _edge_kernel(hhat_hbm, z2_hbm, rowc_hbm, colc_hbm, out_hbm,
                 ridx_v, cidx_v, rows_v, sem_i, sem_g, sem_s, acc_sh):
    c = lax.axis_index("c")
    s = lax.axis_index("s")
    wid = s * 2 + c
    sl = 640

    @pl.when(c == 0)
    def _():
        pltpu.sync_copy(hhat_hbm.at[pl.ds(s * sl, sl)],
                        acc_sh.at[pl.ds(s * sl, sl)])

    @pl.when(c == 1)
    def _():
        pltpu.sync_copy(z2_hbm.at[pl.ds(s * sl, sl)],
                        acc_sh.at[pl.ds(s * sl, sl)])

    plsc.subcore_barrier()

    def _ridx(g4, p):
        return ridx_v.at[g4, p]

    def _cidx(g4, p):
        return cidx_v.at[g4, p]

    def _pf_idx(g, g4):
        pltpu.async_copy(rowc_hbm.at[wid, g], ridx_v.at[g4], sem_i.at[g4])
        pltpu.async_copy(colc_hbm.at[wid, g], cidx_v.at[g4], sem_i.at[g4])

    def _pf_idx_wait(g, g4):
        pltpu.make_async_copy(rowc_hbm.at[wid, g], ridx_v.at[g4],
                              sem_i.at[g4]).wait()
        pltpu.make_async_copy(colc_hbm.at[wid, g], cidx_v.at[g4],
                              sem_i.at[g4]).wait()

    def _gather(g4, p, b):
        pltpu.async_copy(hhat_hbm.at[_ridx(g4, p)], rows_v.at[b], sem_g.at[b])

    def _gather_wait(g4, p, b):
        pltpu.make_async_copy(hhat_hbm.at[_ridx(g4, p)], rows_v.at[b],
                              sem_g.at[b]).wait()

    def _scatter(g4, p, b):
        pltpu.async_copy(rows_v.at[b], acc_sh.at[_cidx(g4, p)],
                         sem_s.at[b], add=True)

    def _scatter_wait(g4, p, b):
        pltpu.make_async_copy(rows_v.at[b], acc_sh.at[_cidx(g4, p)],
                              sem_s.at[b]).wait()

    _pf_idx(0, 0)
    _pf_idx(1, 1)
    _pf_idx_wait(0, 0)
    _gather(0, 0, 0)

    def outer(t, carry):
        for u in range(6):
            j = 6 * t + u
            b = u % 3
            p = u % 2
            gq = 3 * t + u // 2
            g4 = lax.rem(gq, NRING)
            if u % 2 == 0:
                gp = gq + 2

                @pl.when(gp < NGRP)
                def _():
                    _pf_idx(gp, lax.rem(gp, NRING))
            else:
                @pl.when(gq + 1 < NGRP)
                def _():
                    _pf_idx_wait(gq + 1, lax.rem(gq + 1, NRING))

            gm = 3 * t + (u - 2) // 2
            pm = u % 2
            bm = (u + 1) % 3

            @pl.when(j >= 2)
            def _():
                _scatter_wait(lax.rem(gm, NRING), pm, bm)

            gn = 3 * t + (u + 1) // 2
            pn = (u + 1) % 2
            bn = (u + 1) % 3

            @pl.when(j + 1 < NCH)
            def _():
                _gather(lax.rem(gn, NRING), pn, bn)

            _gather_wait(g4, p, b)
            _scatter(g4, p, b)
        return carry

    lax.fori_loop(0, NCH // 6, outer, 0)
    _scatter_wait((NGRP - 1) % NRING, 0, (NCH - 2) % 3)
    _scatter_wait((NGRP - 1) % NRING, 1, (NCH - 1) % 3)
    plsc.subcore_barrier()
    pltpu.sync_copy(acc_sh.at[pl.ds(s * sl, sl)], out_hbm.at[c, pl.ds(s * sl, sl)])


def _pre_body(x_ref, da_ref, db_ref, w_ref, hh_ref, dinv_ref):
    dinv = lax.rsqrt(da_ref[...] + db_ref[...] + 1.0)
    hh_ref[...] = jnp.dot(x_ref[...], w_ref[...],
                          preferred_element_type=jnp.float32) * dinv
    dinv_ref[...] = dinv


_pre = pl.pallas_call(
    _pre_body,
    grid=(NP // RB,),
    in_specs=[
        pl.BlockSpec((RB, D), lambda i: (i, 0)),
        pl.BlockSpec((RB, 1), lambda i: (i, 0)),
        pl.BlockSpec((RB, 1), lambda i: (i, 0)),
        pl.BlockSpec((D, D), lambda i: (0, 0)),
    ],
    out_specs=[
        pl.BlockSpec((RB, D), lambda i: (i, 0)),
        pl.BlockSpec((RB, 1), lambda i: (i, 0)),
    ],
    out_shape=[
        jax.ShapeDtypeStruct((NP, D), jnp.float32),
        jax.ShapeDtypeStruct((NP, 1), jnp.float32),
    ],
)


def _mid_body(a0_ref, a1_ref, dinv_ref, b_ref, w_ref, out_ref):
    dinv = dinv_ref[...]
    h = jnp.maximum(dinv * (a0_ref[...] + a1_ref[...]) + b_ref[...], 0.0)
    out_ref[...] = jnp.dot(h, w_ref[...],
                           preferred_element_type=jnp.float32) * dinv


_mid = pl.pallas_call(
    _mid_body,
    grid=(NP // RB,),
    in_specs=[
        pl.BlockSpec((RB, D), lambda i: (i, 0)),
        pl.BlockSpec((RB, D), lambda i: (i, 0)),
        pl.BlockSpec((RB, 1), lambda i: (i, 0)),
        pl.BlockSpec((1, D), lambda i: (0, 0)),
        pl.BlockSpec((D, D), lambda i: (0, 0)),
    ],
    out_specs=pl.BlockSpec((RB, D), lambda i: (i, 0)),
    out_shape=jax.ShapeDtypeStruct((NP, D), jnp.float32),
)


def _fin_body(a0_ref, a1_ref, dinv_ref, b3_ref, batch_ref, sb_ref,
              psw1_ref, psb1_ref, psw2_ref, psb2_ref,
              phw1_ref, phb1_ref, phw2_ref, phb2_ref,
              y_ref, pooled, cnt):
    j = pl.program_id(0)

    @pl.when(j == 0)
    def _():
        pooled[...] = jnp.zeros_like(pooled)
        cnt[...] = jnp.zeros_like(cnt)

    h = jnp.maximum(dinv_ref[...] * (a0_ref[...] + a1_ref[...]) + b3_ref[...],
                    0.0)
    gids = lax.broadcasted_iota(jnp.int32, (NG, RB), 0)
    mf = (batch_ref[...] == gids).astype(jnp.float32)
    pooled[...] += jnp.dot(mf, h, preferred_element_type=jnp.float32)
    cnt[...] += jnp.sum(mf, axis=1, keepdims=True)

    @pl.when(j == NP // RB - 1)
    def _():
        emb = pooled[...] / jnp.maximum(cnt[...], 1.0)
        t = jnp.maximum(
            jnp.dot(emb, psw1_ref[...], preferred_element_type=jnp.float32)
            + psb1_ref[...], 0.0)
        p = jnp.tanh(
            jnp.dot(t, psw2_ref[...], preferred_element_type=jnp.float32)
            + psb2_ref[...])
        p0 = jnp.tanh(
            jnp.dot(jnp.maximum(psb1_ref[...], 0.0), psw2_ref[...],
                    preferred_element_type=jnp.float32) + psb2_ref[...])
        sids = lax.broadcasted_iota(jnp.int32, (NS, NG), 0)
        sm = (sb_ref[...] == sids).astype(jnp.float32)
        ssize = jnp.sum(sm, axis=1, keepdims=True)
        mx = jnp.max(ssize)
        agg = jnp.dot(sm, p, preferred_element_type=jnp.float32) \
            + (mx - ssize) * p0
        t2 = jnp.maximum(
            jnp.dot(agg, phw1_ref[...], preferred_element_type=jnp.float32)
            + phb1_ref[...], 0.0)
        y_ref[...] = jnp.dot(t2, phw2_ref[...],
                             preferred_element_type=jnp.float32) + phb2_ref[...]


_fin = pl.pallas_call(
    _fin_body,
    grid=(NP // RB,),
    in_specs=[
        pl.BlockSpec((RB, D), lambda i: (i, 0)),
        pl.BlockSpec((RB, D), lambda i: (i, 0)),
        pl.BlockSpec((RB, 1), lambda i: (i, 0)),
        pl.BlockSpec((1, D), lambda i: (0, 0)),
        pl.BlockSpec((1, RB), lambda i: (0, i)),
        pl.BlockSpec((1, NG), lambda i: (0, 0)),
        pl.BlockSpec((D, D), lambda i: (0, 0)),
        pl.BlockSpec((1, D), lambda i: (0, 0)),
        pl.BlockSpec((D, D), lambda i: (0, 0)),
        pl.BlockSpec((1, D), lambda i: (0, 0)),
        pl.BlockSpec((D, D), lambda i: (0, 0)),
        pl.BlockSpec((1, D), lambda i: (0, 0)),
        pl.BlockSpec((D, 16), lambda i: (0, 0)),
        pl.BlockSpec((1, 16), lambda i: (0, 0)),
    ],
    out_specs=pl.BlockSpec((NS, 16), lambda i: (0, 0)),
    out_shape=jax.ShapeDtypeStruct((NS, 16), jnp.float32),
    scratch_shapes=[
        pltpu.VMEM((NG, D), jnp.float32),
        pltpu.VMEM((NG, 1), jnp.float32),
    ],
)


def kernel(x, edge_index, batch, set_batch, W1, b1, W2, b2, W3, b3,
           psi_W1, psi_b1, psi_W2, psi_b2, phi_W1, phi_b1, phi_W2, phi_b2):
    f32 = jnp.float32
    ep = NTILES * NCH * CH - E
    pad_row = (jnp.arange(ep, dtype=jnp.int32) % N)
    pad_col = N + (jnp.arange(ep, dtype=jnp.int32) % (NP - N))
    rowc = jnp.concatenate([edge_index[0], pad_row]).reshape(
        NTILES, NGRP, 2, CH)
    colc = jnp.concatenate([edge_index[1], pad_col]).reshape(
        NTILES, NGRP, 2, CH)
    colc3 = colc.reshape(NTILES, NCH, CH)
    ones = jnp.ones((CH,), f32)
    z1 = jnp.zeros((NP,), f32)
    z2 = jnp.zeros((NP, D), f32)
    xpad = jnp.concatenate([x, jnp.zeros((NP - N, D), f32)], axis=0)
    bpad = jnp.concatenate(
        [batch.astype(jnp.int32),
         jnp.full((NP - N,), NG, jnp.int32)]).reshape(1, NP)
    sb = set_batch.astype(jnp.int32).reshape(1, NG)

    degp = _deg_kernel(colc3, ones, z1)
    da = degp[0].reshape(NP, 1)
    db = degp[1].reshape(NP, 1)
    hh, dinv = _pre(xpad, da, db, W1)

    part = _edge_kernel(hh, z2, rowc, colc)
    hh = _mid(part[0], part[1], dinv, b1.reshape(1, D), W2)
    part = _edge_kernel(hh, z2, rowc, colc)
    hh = _mid(part[0], part[1], dinv, b2.reshape(1, D), W3)
    part = _edge_kernel(hh, z2, rowc, colc)

    y = _fin(part[0], part[1], dinv, b3.reshape(1, D), bpad, sb,
             psi_W1, psi_b1.reshape(1, D), psi_W2, psi_b2.reshape(1, D),
             phi_W1, phi_b1.reshape(1, D), phi_W2, phi_b2.reshape(1, 16))
    return y


# speedup vs baseline: 30.8832x; 1.0743x over previous
import functools

import jax
import jax.numpy as jnp
from jax import lax
from jax.experimental import pallas as pl
from jax.experimental.pallas import tpu as pltpu
from jax.experimental.pallas import tpu_sc as plsc

N = 10000
NP = 10240
E = 320000
D = 128
NG = 512
NS = 64
NTILES = 32
NCH = 90
CH = 112
NBUF = 3
NGRP = 45
NRING = 4
RB = 1024

_mesh = plsc.VectorSubcoreMesh(core_axis_name="c", subcore_axis_name="s")


@functools.partial(
    pl.kernel,
    out_type=jax.ShapeDtypeStruct((2, NP), jnp.float32),
    mesh=_mesh,
    scratch_types=[
        pltpu.VMEM((NCH, CH), jnp.int32),
        pltpu.VMEM((CH,), jnp.float32),
        pltpu.VMEM_SHARED((NP,), jnp.float32),
    ],
)
def _deg_kernel(colc_hbm, ones_hbm, z1_hbm, out_hbm, colidx_v, ones_v, acc_sh):
    c = lax.axis_index("c")
    s = lax.axis_index("s")
    wid = s * 2 + c
    sl = 640
    pltpu.sync_copy(ones_hbm, ones_v)
    pltpu.sync_copy(z1_hbm.at[pl.ds(s * sl, sl)], acc_sh.at[pl.ds(s * sl, sl)])
    pltpu.sync_copy(colc_hbm.at[wid], colidx_v)
    plsc.subcore_barrier()

    def body(j, carry):
        pltpu.sync_copy(ones_v, acc_sh.at[colidx_v.at[j]], add=True)
        return carry

    lax.fori_loop(0, NCH, body, 0)
    plsc.subcore_barrier()
    pltpu.sync_copy(acc_sh.at[pl.ds(s * sl, sl)], out_hbm.at[c, pl.ds(s * sl, sl)])


@functools.partial(
    pl.kernel,
    out_type=jax.ShapeDtypeStruct((2, NP, D), jnp.float32),
    mesh=_mesh,
    scratch_types=[
        pltpu.VMEM((NRING, 2, CH), jnp.int32),
        pltpu.VMEM((NRING, 2, CH), jnp.int32),
        pltpu.VMEM((NBUF, CH, D), jnp.float32),
        pltpu.SemaphoreType.DMA((NRING,)),
        pltpu.SemaphoreType.DMA((NBUF,)),
        pltpu.SemaphoreType.DMA((NBUF,)),
        pltpu.VMEM_SHARED((NP, D), jnp.float32),
    ],
)
def _edge_kernel(hhat_hbm, z2_hbm, rowc_hbm, colc_hbm, out_hbm,
                 ridx_v, cidx_v, rows_v, sem_i, sem_g, sem_s, acc_sh):
    c = lax.axis_index("c")
    s = lax.axis_index("s")
    wid = s * 2 + c
    sl = 640

    @pl.when(c == 0)
    def _():
        pltpu.sync_copy(hhat_hbm.at[pl.ds(s * sl, sl)],
                        acc_sh.at[pl.ds(s * sl, sl)])

    @pl.when(c == 1)
    def _():
        pltpu.sync_copy(z2_hbm.at[pl.ds(s * sl, sl)],
                        acc_sh.at[pl.ds(s * sl, sl)])

    plsc.subcore_barrier()

    def _ridx(g4, p):
        return ridx_v.at[g4, p]

    def _cidx(g4, p):
        return cidx_v.at[g4, p]

    def _pf_idx(g, g4):
        pltpu.async_copy(rowc_hbm.at[wid, g], ridx_v.at[g4], sem_i.at[g4])
        pltpu.async_copy(colc_hbm.at[wid, g], cidx_v.at[g4], sem_i.at[g4])

    def _pf_idx_wait(g, g4):
        pltpu.make_async_copy(rowc_hbm.at[wid, g], ridx_v.at[g4],
                              sem_i.at[g4]).wait()
        pltpu.make_async_copy(colc_hbm.at[wid, g], cidx_v.at[g4],
                              sem_i.at[g4]).wait()

    def _gather(g4, p, b):
        pltpu.async_copy(hhat_hbm.at[_ridx(g4, p)], rows_v.at[b], sem_g.at[b])

    def _gather_wait(g4, p, b):
        pltpu.make_async_copy(hhat_hbm.at[_ridx(g4, p)], rows_v.at[b],
                              sem_g.at[b]).wait()

    def _scatter(g4, p, b):
        pltpu.async_copy(rows_v.at[b], acc_sh.at[_cidx(g4, p)],
                         sem_s.at[b], add=True)

    def _scatter_wait(g4, p, b):
        pltpu.make_async_copy(rows_v.at[b], acc_sh.at[_cidx(g4, p)],
                              sem_s.at[b]).wait()

    _pf_idx(0, 0)
    _pf_idx(1, 1)
    _pf_idx_wait(0, 0)
    _gather(0, 0, 0)

    def outer(t, carry):
        for u in range(6):
            j = 6 * t + u
            b = u % 3
            p = u % 2
            gq = 3 * t + u // 2
            g4 = lax.rem(gq, NRING)
            if u % 2 == 0:
                gp = gq + 2

                @pl.when(gp < NGRP)
                def _():
                    _pf_idx(gp, lax.rem(gp, NRING))
            else:
                @pl.when(gq + 1 < NGRP)
                def _():
                    _pf_idx_wait(gq + 1, lax.rem(gq + 1, NRING))

            gm = 3 * t + (u - 2) // 2
            pm = u % 2
            bm = (u + 1) % 3

            @pl.when(j >= 2)
            def _():
                _scatter_wait(lax.rem(gm, NRING), pm, bm)

            gn = 3 * t + (u + 1) // 2
            pn = (u + 1) % 2
            bn = (u + 1) % 3

            @pl.when(j + 1 < NCH)
            def _():
                _gather(lax.rem(gn, NRING), pn, bn)

            _gather_wait(g4, p, b)
            _scatter(g4, p, b)
        return carry

    lax.fori_loop(0, NCH // 6, outer, 0)
    _scatter_wait((NGRP - 1) % NRING, 0, (NCH - 2) % 3)
    _scatter_wait((NGRP - 1) % NRING, 1, (NCH - 1) % 3)
    plsc.subcore_barrier()
    pltpu.sync_copy(acc_sh.at[pl.ds(s * sl, sl)], out_hbm.at[c, pl.ds(s * sl, sl)])


def _colvec(d2):
    return d2[:, :, None]


def _rows3(a):
    return a.reshape(RB // 128, 128, D)


def _pre_body(x_ref, dg_ref, w_ref, hh_ref, dinv_ref):
    dinv2 = lax.rsqrt(dg_ref[0] + dg_ref[1] + 1.0)
    mm = jnp.dot(x_ref[...], w_ref[...], preferred_element_type=jnp.float32)
    hh_ref[...] = (_rows3(mm) * _colvec(dinv2)).reshape(RB, D)
    dinv_ref[...] = dinv2


_pre = pl.pallas_call(
    _pre_body,
    grid=(NP // RB,),
    in_specs=[
        pl.BlockSpec((RB, D), lambda i: (i, 0)),
        pl.BlockSpec((2, RB // 128, 128), lambda i: (0, i, 0)),
        pl.BlockSpec((D, D), lambda i: (0, 0)),
    ],
    out_specs=[
        pl.BlockSpec((RB, D), lambda i: (i, 0)),
        pl.BlockSpec((RB // 128, 128), lambda i: (i, 0)),
    ],
    out_shape=[
        jax.ShapeDtypeStruct((NP, D), jnp.float32),
        jax.ShapeDtypeStruct((NP // 128, 128), jnp.float32),
    ],
)


def _mid_body(a0_ref, a1_ref, dinv_ref, b_ref, w_ref, out_ref):
    dv = _colvec(dinv_ref[...])
    a = _rows3(a0_ref[0] + a1_ref[0]) * dv
    h = jnp.maximum(a.reshape(RB, D) + b_ref[...], 0.0)
    mm = jnp.dot(h, w_ref[...], preferred_element_type=jnp.float32)
    out_ref[...] = (_rows3(mm) * dv).reshape(RB, D)


_mid = pl.pallas_call(
    _mid_body,
    grid=(NP // RB,),
    in_specs=[
        pl.BlockSpec((1, RB, D), lambda i: (0, i, 0)),
        pl.BlockSpec((1, RB, D), lambda i: (1, i, 0)),
        pl.BlockSpec((RB // 128, 128), lambda i: (i, 0)),
        pl.BlockSpec((1, D), lambda i: (0, 0)),
        pl.BlockSpec((D, D), lambda i: (0, 0)),
    ],
    out_specs=pl.BlockSpec((RB, D), lambda i: (i, 0)),
    out_shape=jax.ShapeDtypeStruct((NP, D), jnp.float32),
)


def _fin_body(a0_ref, a1_ref, dinv_ref, b3_ref, batch_ref, sb_ref,
              psw1_ref, psb1_ref, psw2_ref, psb2_ref,
              phw1_ref, phb1_ref, phw2_ref, phb2_ref,
              y_ref, pooled, cnt):
    j = pl.program_id(0)

    @pl.when(j == 0)
    def _():
        pooled[...] = jnp.zeros_like(pooled)
        cnt[...] = jnp.zeros_like(cnt)

    a = _rows3(a0_ref[0] + a1_ref[0]) * _colvec(dinv_ref[...])
    h = jnp.maximum(a.reshape(RB, D) + b3_ref[...], 0.0)
    gids = lax.broadcasted_iota(jnp.int32, (NG, RB), 0)
    mf = (batch_ref[...] == gids).astype(jnp.float32)
    pooled[...] += jnp.dot(mf, h, preferred_element_type=jnp.float32)
    cnt[...] += jnp.sum(mf, axis=1, keepdims=True)

    @pl.when(j == NP // RB - 1)
    def _():
        emb = pooled[...] / jnp.maximum(cnt[...], 1.0)
        t = jnp.maximum(
            jnp.dot(emb, psw1_ref[...], preferred_element_type=jnp.float32)
            + psb1_ref[...], 0.0)
        p = jnp.tanh(
            jnp.dot(t, psw2_ref[...], preferred_element_type=jnp.float32)
            + psb2_ref[...])
        p0 = jnp.tanh(
            jnp.dot(jnp.maximum(psb1_ref[...], 0.0), psw2_ref[...],
                    preferred_element_type=jnp.float32) + psb2_ref[...])
        sids = lax.broadcasted_iota(jnp.int32, (NS, NG), 0)
        sm = (sb_ref[...] == sids).astype(jnp.float32)
        ssize = jnp.sum(sm, axis=1, keepdims=True)
        mx = jnp.max(ssize)
        agg = jnp.dot(sm, p, preferred_element_type=jnp.float32) \
            + (mx - ssize) * p0
        t2 = jnp.maximum(
            jnp.dot(agg, phw1_ref[...], preferred_element_type=jnp.float32)
            + phb1_ref[...], 0.0)
        y_ref[...] = jnp.dot(t2, phw2_ref[...],
                             preferred_element_type=jnp.float32) + phb2_ref[...]


_fin = pl.pallas_call(
    _fin_body,
    grid=(NP // RB,),
    in_specs=[
        pl.BlockSpec((1, RB, D), lambda i: (0, i, 0)),
        pl.BlockSpec((1, RB, D), lambda i: (1, i, 0)),
        pl.BlockSpec((RB // 128, 128), lambda i: (i, 0)),
        pl.BlockSpec((1, D), lambda i: (0, 0)),
        pl.BlockSpec((1, RB), lambda i: (0, i)),
        pl.BlockSpec((1, NG), lambda i: (0, 0)),
        pl.BlockSpec((D, D), lambda i: (0, 0)),
        pl.BlockSpec((1, D), lambda i: (0, 0)),
        pl.BlockSpec((D, D), lambda i: (0, 0)),
        pl.BlockSpec((1, D), lambda i: (0, 0)),
        pl.BlockSpec((D, D), lambda i: (0, 0)),
        pl.BlockSpec((1, D), lambda i: (0, 0)),
        pl.BlockSpec((D, 16), lambda i: (0, 0)),
        pl.BlockSpec((1, 16), lambda i: (0, 0)),
    ],
    out_specs=pl.BlockSpec((NS, 16), lambda i: (0, 0)),
    out_shape=jax.ShapeDtypeStruct((NS, 16), jnp.float32),
    scratch_shapes=[
        pltpu.VMEM((NG, D), jnp.float32),
        pltpu.VMEM((NG, 1), jnp.float32),
    ],
)


def kernel(x, edge_index, batch, set_batch, W1, b1, W2, b2, W3, b3,
           psi_W1, psi_b1, psi_W2, psi_b2, phi_W1, phi_b1, phi_W2, phi_b2):
    f32 = jnp.float32
    ep = NTILES * NCH * CH - E
    pad_row = (jnp.arange(ep, dtype=jnp.int32) % N)
    pad_col = N + (jnp.arange(ep, dtype=jnp.int32) % (NP - N))
    rowc = jnp.concatenate([edge_index[0], pad_row]).reshape(
        NTILES, NGRP, 2, CH)
    colc = jnp.concatenate([edge_index[1], pad_col]).reshape(
        NTILES, NGRP, 2, CH)
    colc3 = colc.reshape(NTILES, NCH, CH)
    ones = jnp.ones((CH,), f32)
    z1 = jnp.zeros((NP,), f32)
    z2 = jnp.zeros((NP, D), f32)
    xpad = jnp.concatenate([x, jnp.zeros((NP - N, D), f32)], axis=0)
    bpad = jnp.concatenate(
        [batch.astype(jnp.int32),
         jnp.full((NP - N,), NG, jnp.int32)]).reshape(1, NP)
    sb = set_batch.astype(jnp.int32).reshape(1, NG)

    degp = _deg_kernel(colc3, ones, z1).reshape(2, NP // 128, 128)
    hh, dinv = _pre(xpad, degp, W1)

    part = _edge_kernel(hh, z2, rowc, colc)
    hh = _mid(part, part, dinv, b1.reshape(1, D), W2)
    part = _edge_kernel(hh, z2, rowc, colc)
    hh = _mid(part, part, dinv, b2.reshape(1, D), W3)
    part = _edge_kernel(hh, z2, rowc, colc)

    y = _fin(part, part, dinv, b3.reshape(1, D), bpad, sb,
             psi_W1, psi_b1.reshape(1, D), psi_W2, psi_b2.reshape(1, D),
             phi_W1, phi_b1.reshape(1, D), phi_W2, phi_b2.reshape(1, 16))
    return y
